# baseline (speedup 1.0000x reference)
import functools

import jax
import jax.numpy as jnp
from jax import lax
from jax.experimental import pallas as pl
from jax.experimental.pallas import tpu as pltpu
from jax.experimental.pallas import tpu_sc as plsc

N_NODES = 10000
N_PAD = 10240
N_EDGES = 320000
NSD = 32
ESD = 16
MSG = 32
GREP = 128
HEAD = 256
N_GRAPHS = 16

_NW = 32
_CHUNK = 128
_CPW = 80
E_PAD = _NW * _CPW * _CHUNK


@functools.cache
def _build_gather():
    mesh = plsc.VectorSubcoreMesh(
        core_axis_name="c", subcore_axis_name="s", num_cores=2, num_subcores=16)

    @functools.partial(
        pl.kernel,
        out_type=(jax.ShapeDtypeStruct((E_PAD, NSD), jnp.float32),
                  jax.ShapeDtypeStruct((E_PAD, NSD), jnp.float32)),
        mesh=mesh,
        scratch_types=[
            pltpu.VMEM((_CHUNK,), jnp.int32),
            pltpu.VMEM((_CHUNK,), jnp.int32),
            pltpu.VMEM((_CHUNK, NSD), jnp.float32),
            pltpu.VMEM((_CHUNK, NSD), jnp.float32),
            pltpu.VMEM_SHARED((N_PAD, NSD), jnp.float32),
            pltpu.SemaphoreType.DMA,
            pltpu.SemaphoreType.DMA,
        ],
        compiler_params=pltpu.CompilerParams(use_tc_tiling_on_sc=False),
    )
    def gather_edges(h_hbm, fidx_hbm, tidx_hbm, src_out, dst_out,
                     fidx_v, tidx_v, srow_v, trow_v, h_sh, sem_f, sem_t):
        cid = lax.axis_index("c")
        sid = lax.axis_index("s")
        wid = sid * 2 + cid

        @pl.when(sid == 0)
        def _():
            pltpu.sync_copy(h_hbm, h_sh)

        plsc.subcore_barrier()

        def body(j, carry):
            base = (wid * _CPW + j) * _CHUNK
            pltpu.sync_copy(fidx_hbm.at[pl.ds(base, _CHUNK)], fidx_v)
            pltpu.sync_copy(tidx_hbm.at[pl.ds(base, _CHUNK)], tidx_v)
            cp_f = pltpu.async_copy(h_sh.at[fidx_v], srow_v, sem_f)
            cp_t = pltpu.async_copy(h_sh.at[tidx_v], trow_v, sem_t)
            cp_f.wait()
            cp_t.wait()
            pltpu.sync_copy(srow_v, src_out.at[pl.ds(base, _CHUNK)])
            pltpu.sync_copy(trow_v, dst_out.at[pl.ds(base, _CHUNK)])
            return carry

        lax.fori_loop(0, _CPW, body, 0)

    return gather_edges


def _gather_edges(h, fi, ti):
    return _build_gather()(h, fi, ti)


@functools.cache
def _build_scatter():
    mesh = plsc.VectorSubcoreMesh(
        core_axis_name="c", subcore_axis_name="s", num_cores=2, num_subcores=16)

    @functools.partial(
        pl.kernel,
        out_type=jax.ShapeDtypeStruct((2, N_PAD, MSG), jnp.float32),
        mesh=mesh,
        scratch_types=[
            pltpu.VMEM((_CPW, _CHUNK), jnp.int32),
            pltpu.VMEM((_CHUNK, MSG), jnp.float32),
            pltpu.VMEM_SHARED((N_PAD, MSG), jnp.float32),
        ],
        compiler_params=pltpu.CompilerParams(use_tc_tiling_on_sc=False),
    )
    def scatter_msg(msg_hbm, tidx2d_hbm, zeros_hbm, out_hbm, idx_v, msg_v, acc_sh):
        cid = lax.axis_index("c")
        sid = lax.axis_index("s")
        wid = sid * 2 + cid

        @pl.when(sid == 0)
        def _():
            pltpu.sync_copy(zeros_hbm, acc_sh)

        plsc.subcore_barrier()
        pltpu.sync_copy(tidx2d_hbm.at[pl.ds(wid * _CPW, _CPW)], idx_v)

        def body(j, carry):
            base = (wid * _CPW + j) * _CHUNK
            pltpu.sync_copy(msg_hbm.at[pl.ds(base, _CHUNK)], msg_v)
            pltpu.sync_copy(msg_v, acc_sh.at[idx_v.at[j]], add=True)
            return carry

        lax.fori_loop(0, _CPW, body, 0)
        plsc.subcore_barrier()

        @pl.when(sid == 0)
        def _():
            pltpu.sync_copy(acc_sh, out_hbm.at[cid])

    return scatter_msg


def _scatter_msg(msg, ti2d, zeros_nm):
    return _build_scatter()(msg, ti2d, zeros_nm)


def _const2d(shape):
    return pl.BlockSpec(shape, lambda i: (0, 0))


def _encoder_body(x_ref, w_ref, b_ref, o_ref):
    o_ref[...] = jnp.dot(x_ref[...], w_ref[...],
                         preferred_element_type=jnp.float32) + b_ref[...]


def _node_encoder(x, w, b):
    blk = 1024
    return pl.pallas_call(
        _encoder_body,
        grid=(N_PAD // blk,),
        in_specs=[pl.BlockSpec((blk, 128), lambda i: (i, 0)),
                  _const2d((128, NSD)), _const2d((1, NSD))],
        out_specs=pl.BlockSpec((blk, NSD), lambda i: (i, 0)),
        out_shape=jax.ShapeDtypeStruct((N_PAD, NSD), jnp.float32),
    )(x, w, b)


def _edge_mlp_body(src_ref, dst_ref, x_ref, wee_ref, bee_ref,
                   a1_ref, a2_ref, c1_ref, b1_ref, w2_ref, b2_ref,
                   ae_ref, be_ref, ce_ref, beu_ref, msg_ref, eo_ref):
    f32 = jnp.float32
    src = src_ref[...]
    dst = dst_ref[...]
    e0 = jnp.dot(x_ref[...], wee_ref[...], preferred_element_type=f32) + bee_ref[...]
    hid = (jnp.dot(src, a1_ref[...], preferred_element_type=f32)
           + jnp.dot(dst, a2_ref[...], preferred_element_type=f32)
           + jnp.dot(e0, c1_ref[...], preferred_element_type=f32)
           + b1_ref[...])
    hid = jnp.maximum(hid, 0.0)
    msg_ref[...] = jnp.dot(hid, w2_ref[...], preferred_element_type=f32) + b2_ref[...]
    eo = (jnp.dot(src, ae_ref[...], preferred_element_type=f32)
          + jnp.dot(dst, be_ref[...], preferred_element_type=f32)
          + jnp.dot(e0, ce_ref[...], preferred_element_type=f32)
          + beu_ref[...])
    eo_ref[...] = jnp.maximum(eo, 0.0)


def _edge_mlp(src, dst, x, wee, bee, a1, a2, c1, b1, w2, b2, ae, be, ce, beu):
    blk = 1024
    return pl.pallas_call(
        _edge_mlp_body,
        grid=(E_PAD // blk,),
        in_specs=[pl.BlockSpec((blk, NSD), lambda i: (i, 0)),
                  pl.BlockSpec((blk, NSD), lambda i: (i, 0)),
                  pl.BlockSpec((blk, ESD), lambda i: (i, 0)),
                  _const2d((ESD, ESD)), _const2d((1, ESD)),
                  _const2d((NSD, 64)), _const2d((NSD, 64)),
                  _const2d((ESD, 64)), _const2d((1, 64)),
                  _const2d((64, MSG)), _const2d((1, MSG)),
                  _const2d((NSD, ESD)), _const2d((NSD, ESD)),
                  _const2d((ESD, ESD)), _const2d((1, ESD))],
        out_specs=[pl.BlockSpec((blk, MSG), lambda i: (i, 0)),
                   pl.BlockSpec((blk, ESD), lambda i: (i, 0))],
        out_shape=[jax.ShapeDtypeStruct((E_PAD, MSG), jnp.float32),
                   jax.ShapeDtypeStruct((E_PAD, ESD), jnp.float32)],
    )(src, dst, x, wee, bee, a1, a2, c1, b1, w2, b2, ae, be, ce, beu)


_ROWS = 256


def _attn_update_body(h_ref, gcol_ref, grow_ref, agg0_ref, agg1_ref,
                      u1h_ref, u1a_ref, u1c_ref, b1_ref, w2_ref, b2_ref,
                      o_ref):
    f32 = jnp.float32
    i = pl.program_id(0)
    h_all = h_ref[...]
    hr = h_ref[pl.ds(i * _ROWS, _ROWS), :]
    mask = (grow_ref[...] ^ 1) == gcol_ref[...]
    scores = lax.dot_general(hr, h_all, (((1,), (1,)), ((), ())),
                             preferred_element_type=f32)
    scores = jnp.where(mask, scores, jnp.float32(-1e9))
    mx = jnp.max(scores, axis=1, keepdims=True)
    p = jnp.exp(scores - mx)
    denom = jnp.sum(p, axis=1, keepdims=True)
    att = p / denom
    cross = hr - lax.dot_general(att, h_all, (((1,), (0,)), ((), ())),
                                 preferred_element_type=f32)
    agg = agg0_ref[...] + agg1_ref[...]
    u = (jnp.dot(hr, u1h_ref[...], preferred_element_type=f32)
         + jnp.dot(agg, u1a_ref[...], preferred_element_type=f32)
         + jnp.dot(cross, u1c_ref[...], preferred_element_type=f32)
         + b1_ref[...])
    u = jnp.maximum(u, 0.0)
    o_ref[...] = jnp.dot(u, w2_ref[...], preferred_element_type=f32) + b2_ref[...]


def _attn_update(h, gcol, grow, agg0, agg1, u1h, u1a, u1c, b1, w2, b2):
    return pl.pallas_call(
        _attn_update_body,
        grid=(N_PAD // _ROWS,),
        in_specs=[_const2d((N_PAD, NSD)),
                  _const2d((1, N_PAD)),
                  pl.BlockSpec((_ROWS, 1), lambda i: (i, 0)),
                  pl.BlockSpec((_ROWS, MSG), lambda i: (i, 0)),
                  pl.BlockSpec((_ROWS, MSG), lambda i: (i, 0)),
                  _const2d((NSD, 64)), _const2d((MSG, 64)),
                  _const2d((NSD, 64)), _const2d((1, 64)),
                  _const2d((64, NSD)), _const2d((1, NSD))],
        out_specs=pl.BlockSpec((_ROWS, NSD), lambda i: (i, 0)),
        out_shape=jax.ShapeDtypeStruct((N_PAD, NSD), jnp.float32),
    )(h, gcol, grow, agg0, agg1, u1h, u1a, u1c, b1, w2, b2)


def _readout_body(h_ref, grow_ref, wg_ref, bg_ref, wv_ref, bv_ref,
                  w1a_ref, w1b_ref, b1_ref, w2_ref, b2_ref, se_ref, so_ref,
                  rep_ref, out_ref):
    f32 = jnp.float32
    i = pl.program_id(0)
    hb = h_ref[...]
    gate_z = jnp.dot(hb, wg_ref[...], preferred_element_type=f32) + bg_ref[...]
    gate = 1.0 / (1.0 + jnp.exp(-gate_z))
    val = jnp.dot(hb, wv_ref[...], preferred_element_type=f32) + bv_ref[...]
    gv = gate * val
    gid = lax.broadcasted_iota(jnp.int32, (1, N_GRAPHS), 1)
    onehot = (grow_ref[...] == gid).astype(f32)
    contrib = lax.dot_general(onehot, gv, (((0,), (0,)), ((), ())),
                              preferred_element_type=f32)

    @pl.when(i == 0)
    def _():
        rep_ref[...] = contrib

    @pl.when(i > 0)
    def _():
        rep_ref[...] = rep_ref[...] + contrib

    @pl.when(i == pl.num_programs(0) - 1)
    def _():
        rep = rep_ref[...]
        t = (jnp.dot(se_ref[...],
                     jnp.dot(rep, w1a_ref[...], preferred_element_type=f32),
                     preferred_element_type=f32)
             + jnp.dot(so_ref[...],
                       jnp.dot(rep, w1b_ref[...], preferred_element_type=f32),
                       preferred_element_type=f32)
             + b1_ref[...])
        t = jnp.maximum(t, 0.0)
        out_ref[...] = jnp.dot(t, w2_ref[...], preferred_element_type=f32) + b2_ref[...]


def _readout(h, grow, wg, bg, wv, bv, w1a, w1b, b1, w2, b2, se, so):
    n_pairs = N_GRAPHS // 2
    return pl.pallas_call(
        _readout_body,
        grid=(N_PAD // _ROWS,),
        in_specs=[pl.BlockSpec((_ROWS, NSD), lambda i: (i, 0)),
                  pl.BlockSpec((_ROWS, 1), lambda i: (i, 0)),
                  _const2d((NSD, GREP)), _const2d((1, GREP)),
                  _const2d((NSD, GREP)), _const2d((1, GREP)),
                  _const2d((GREP, HEAD)), _const2d((GREP, HEAD)),
                  _const2d((1, HEAD)), _const2d((HEAD, 1)), _const2d((1, 1)),
                  _const2d((n_pairs, N_GRAPHS)), _const2d((n_pairs, N_GRAPHS))],
        out_specs=[_const2d((N_GRAPHS, GREP)),
                   _const2d((n_pairs, 1))],
        out_shape=[jax.ShapeDtypeStruct((N_GRAPHS, GREP), jnp.float32),
                   jax.ShapeDtypeStruct((n_pairs, 1), jnp.float32)],
    )(h, grow, wg, bg, wv, bv, w1a, w1b, b1, w2, b2, se, so)


def kernel(node_features, edge_features, from_idx, to_idx, graph_idx,
           graph_idx_4edge, training_n_graphs_in_batch,
           W_ne, b_ne, W_ee, b_ee, W_m1, b_m1, W_m2, b_m2, W_eu, b_eu,
           W_u1, b_u1, W_u2, b_u2, W_g, b_g, W_v, b_v,
           W_fc1, b_fc1, W_fc2, b_fc2):
    f32 = jnp.float32
    ep = E_PAD - N_EDGES
    nf = jnp.pad(node_features, ((0, N_PAD - N_NODES), (0, 0)))
    ef = jnp.pad(edge_features, ((0, ep), (0, 0)))
    fi = jnp.pad(from_idx, (0, ep), constant_values=N_NODES)
    ti = jnp.pad(to_idx, (0, ep), constant_values=N_NODES)
    gi = jnp.pad(graph_idx, (0, N_PAD - N_NODES), constant_values=N_GRAPHS)
    gcol = gi.reshape(1, N_PAD)
    grow = gi.reshape(N_PAD, 1)
    ti2d = ti.reshape(E_PAD // _CHUNK, _CHUNK)
    zeros_nm = jnp.zeros((N_PAD, MSG), f32)

    a1, a2, c1 = W_m1[:NSD], W_m1[NSD:2 * NSD], W_m1[2 * NSD:]
    ae, be, ce = W_eu[:NSD], W_eu[NSD:2 * NSD], W_eu[2 * NSD:]
    u1h, u1a, u1c = W_u1[:NSD], W_u1[NSD:NSD + MSG], W_u1[NSD + MSG:]
    w1a, w1b = W_fc1[:GREP], W_fc1[GREP:]
    b_ne2 = b_ne.reshape(1, -1)
    b_ee2 = b_ee.reshape(1, -1)
    b_m12 = b_m1.reshape(1, -1)
    b_m22 = b_m2.reshape(1, -1)
    b_eu2 = b_eu.reshape(1, -1)
    b_u12 = b_u1.reshape(1, -1)
    b_u22 = b_u2.reshape(1, -1)
    b_g2 = b_g.reshape(1, -1)
    b_v2 = b_v.reshape(1, -1)
    b_fc12 = b_fc1.reshape(1, -1)
    b_fc22 = b_fc2.reshape(1, -1)
    eye16 = jnp.eye(N_GRAPHS, dtype=f32)
    se, so = eye16[0::2], eye16[1::2]
    eye_e = jnp.eye(ESD, dtype=f32)
    zero_e = jnp.zeros((1, ESD), f32)

    h = _node_encoder(nf, W_ne, b_ne2)
    e = ef
    wee, bee = W_ee, b_ee2
    for _ in range(2):
        src, dst = _gather_edges(h, fi, ti)
        msg, e = _edge_mlp(src, dst, e, wee, bee, a1, a2, c1, b_m12,
                           W_m2, b_m22, ae, be, ce, b_eu2)
        wee, bee = eye_e, zero_e
        parts = _scatter_msg(msg, ti2d, zeros_nm)
        h = _attn_update(h, gcol, grow, parts[0], parts[1],
                         u1h, u1a, u1c, b_u12, W_u2, b_u22)

    _, out = _readout(h, grow, W_g, b_g2, W_v, b_v2,
                      w1a, w1b, b_fc12, W_fc2, b_fc22, se, so)
    return out[:, 0]


# speedup vs baseline: 1.1980x; 1.1980x over previous
import functools

import jax
import jax.numpy as jnp
from jax import lax
from jax.experimental import pallas as pl
from jax.experimental.pallas import tpu as pltpu
from jax.experimental.pallas import tpu_sc as plsc

N_NODES = 10000
N_PAD = 10240
N_EDGES = 320000
NSD = 32
ESD = 16
MSG = 32
GREP = 128
HEAD = 256
N_GRAPHS = 16

_NW = 32
_CHUNK = 128
_CPW = 80
E_PAD = _NW * _CPW * _CHUNK


@functools.cache
def _build_gather():
    mesh = plsc.VectorSubcoreMesh(
        core_axis_name="c", subcore_axis_name="s", num_cores=2, num_subcores=16)

    @functools.partial(
        pl.kernel,
        out_type=(jax.ShapeDtypeStruct((E_PAD, NSD), jnp.float32),
                  jax.ShapeDtypeStruct((E_PAD, NSD), jnp.float32)),
        mesh=mesh,
        scratch_types=[
            pltpu.VMEM((_CHUNK,), jnp.int32),
            pltpu.VMEM((_CHUNK,), jnp.int32),
            pltpu.VMEM((_CHUNK, NSD), jnp.float32),
            pltpu.VMEM((_CHUNK, NSD), jnp.float32),
            pltpu.VMEM_SHARED((N_PAD, NSD), jnp.float32),
            pltpu.SemaphoreType.DMA,
            pltpu.SemaphoreType.DMA,
        ],
        compiler_params=pltpu.CompilerParams(use_tc_tiling_on_sc=False),
    )
    def gather_edges(h_hbm, fidx_hbm, tidx_hbm, src_out, dst_out,
                     fidx_v, tidx_v, srow_v, trow_v, h_sh, sem_f, sem_t):
        cid = lax.axis_index("c")
        sid = lax.axis_index("s")
        wid = sid * 2 + cid

        @pl.when(sid == 0)
        def _():
            pltpu.sync_copy(h_hbm, h_sh)

        plsc.subcore_barrier()

        def body(j, carry):
            base = (wid * _CPW + j) * _CHUNK
            pltpu.sync_copy(fidx_hbm.at[pl.ds(base, _CHUNK)], fidx_v)
            pltpu.sync_copy(tidx_hbm.at[pl.ds(base, _CHUNK)], tidx_v)
            cp_f = pltpu.async_copy(h_sh.at[fidx_v], srow_v, sem_f)
            cp_t = pltpu.async_copy(h_sh.at[tidx_v], trow_v, sem_t)
            cp_f.wait()
            cp_t.wait()
            pltpu.sync_copy(srow_v, src_out.at[pl.ds(base, _CHUNK)])
            pltpu.sync_copy(trow_v, dst_out.at[pl.ds(base, _CHUNK)])
            return carry

        lax.fori_loop(0, _CPW, body, 0)

    return gather_edges


def _gather_edges(h, fi, ti):
    return _build_gather()(h, fi, ti)


@functools.cache
def _build_scatter():
    mesh = plsc.VectorSubcoreMesh(
        core_axis_name="c", subcore_axis_name="s", num_cores=2, num_subcores=16)

    @functools.partial(
        pl.kernel,
        out_type=jax.ShapeDtypeStruct((2, N_PAD, MSG), jnp.float32),
        mesh=mesh,
        scratch_types=[
            pltpu.VMEM((_CPW, _CHUNK), jnp.int32),
            pltpu.VMEM((_CHUNK, MSG), jnp.float32),
            pltpu.VMEM_SHARED((N_PAD, MSG), jnp.float32),
        ],
        compiler_params=pltpu.CompilerParams(use_tc_tiling_on_sc=False),
    )
    def scatter_msg(msg_hbm, tidx2d_hbm, zeros_hbm, out_hbm, idx_v, msg_v, acc_sh):
        cid = lax.axis_index("c")
        sid = lax.axis_index("s")
        wid = sid * 2 + cid

        @pl.when(sid == 0)
        def _():
            pltpu.sync_copy(zeros_hbm, acc_sh)

        plsc.subcore_barrier()
        pltpu.sync_copy(tidx2d_hbm.at[pl.ds(wid * _CPW, _CPW)], idx_v)

        def body(j, carry):
            base = (wid * _CPW + j) * _CHUNK
            pltpu.sync_copy(msg_hbm.at[pl.ds(base, _CHUNK)], msg_v)
            pltpu.sync_copy(msg_v, acc_sh.at[idx_v.at[j]], add=True)
            return carry

        lax.fori_loop(0, _CPW, body, 0)
        plsc.subcore_barrier()

        @pl.when(sid == 0)
        def _():
            pltpu.sync_copy(acc_sh, out_hbm.at[cid])

    return scatter_msg


def _scatter_msg(msg, ti2d, zeros_nm):
    return _build_scatter()(msg, ti2d, zeros_nm)


def _const2d(shape):
    return pl.BlockSpec(shape, lambda i: (0, 0))


def _encoder_body(x_ref, w_ref, b_ref, o_ref):
    o_ref[...] = jnp.dot(x_ref[...], w_ref[...],
                         preferred_element_type=jnp.float32) + b_ref[...]


def _node_encoder(x, w, b):
    blk = 2048
    return pl.pallas_call(
        _encoder_body,
        grid=(N_PAD // blk,),
        in_specs=[pl.BlockSpec((blk, 128), lambda i: (i, 0)),
                  _const2d((128, NSD)), _const2d((1, NSD))],
        out_specs=pl.BlockSpec((blk, NSD), lambda i: (i, 0)),
        out_shape=jax.ShapeDtypeStruct((N_PAD, NSD), jnp.float32),
    )(x, w, b)


def _edge_mlp_body(src_ref, dst_ref, x_ref, wee_ref, bee_ref,
                   a1_ref, a2_ref, c1_ref, b1_ref, w2_ref, b2_ref,
                   ae_ref, be_ref, ce_ref, beu_ref, msg_ref, eo_ref):
    f32 = jnp.float32
    src = src_ref[...]
    dst = dst_ref[...]
    e0 = jnp.dot(x_ref[...], wee_ref[...], preferred_element_type=f32) + bee_ref[...]
    hid = (jnp.dot(src, a1_ref[...], preferred_element_type=f32)
           + jnp.dot(dst, a2_ref[...], preferred_element_type=f32)
           + jnp.dot(e0, c1_ref[...], preferred_element_type=f32)
           + b1_ref[...])
    hid = jnp.maximum(hid, 0.0)
    msg_ref[...] = jnp.dot(hid, w2_ref[...], preferred_element_type=f32) + b2_ref[...]
    eo = (jnp.dot(src, ae_ref[...], preferred_element_type=f32)
          + jnp.dot(dst, be_ref[...], preferred_element_type=f32)
          + jnp.dot(e0, ce_ref[...], preferred_element_type=f32)
          + beu_ref[...])
    eo_ref[...] = jnp.maximum(eo, 0.0)


def _edge_mlp(src, dst, x, wee, bee, a1, a2, c1, b1, w2, b2, ae, be, ce, beu):
    blk = 8192
    return pl.pallas_call(
        _edge_mlp_body,
        grid=(E_PAD // blk,),
        in_specs=[pl.BlockSpec((blk, NSD), lambda i: (i, 0)),
                  pl.BlockSpec((blk, NSD), lambda i: (i, 0)),
                  pl.BlockSpec((blk, ESD), lambda i: (i, 0)),
                  _const2d((ESD, ESD)), _const2d((1, ESD)),
                  _const2d((NSD, 64)), _const2d((NSD, 64)),
                  _const2d((ESD, 64)), _const2d((1, 64)),
                  _const2d((64, MSG)), _const2d((1, MSG)),
                  _const2d((NSD, ESD)), _const2d((NSD, ESD)),
                  _const2d((ESD, ESD)), _const2d((1, ESD))],
        out_specs=[pl.BlockSpec((blk, MSG), lambda i: (i, 0)),
                   pl.BlockSpec((blk, ESD), lambda i: (i, 0))],
        out_shape=[jax.ShapeDtypeStruct((E_PAD, MSG), jnp.float32),
                   jax.ShapeDtypeStruct((E_PAD, ESD), jnp.float32)],
    )(src, dst, x, wee, bee, a1, a2, c1, b1, w2, b2, ae, be, ce, beu)


_ROWS = 128


def _attn_update_body(h_ref, gcol_ref, grow_ref, agg0_ref, agg1_ref,
                      u1h_ref, u1a_ref, u1c_ref, b1_ref, w2_ref, b2_ref,
                      o_ref):
    f32 = jnp.float32
    i = pl.program_id(0)
    h_all = h_ref[...]
    hr = h_ref[pl.ds(i * _ROWS, _ROWS), :]
    mask = (grow_ref[...] ^ 1) == gcol_ref[...]
    scores = lax.dot_general(hr, h_all, (((1,), (1,)), ((), ())),
                             preferred_element_type=f32)
    scores = jnp.where(mask, scores, jnp.float32(-1e9))
    mx = jnp.max(scores, axis=1, keepdims=True)
    p = jnp.exp(scores - mx)
    denom = jnp.sum(p, axis=1, keepdims=True)
    att = p / denom
    cross = hr - lax.dot_general(att, h_all, (((1,), (0,)), ((), ())),
                                 preferred_element_type=f32)
    agg = agg0_ref[...] + agg1_ref[...]
    u = (jnp.dot(hr, u1h_ref[...], preferred_element_type=f32)
         + jnp.dot(agg, u1a_ref[...], preferred_element_type=f32)
         + jnp.dot(cross, u1c_ref[...], preferred_element_type=f32)
         + b1_ref[...])
    u = jnp.maximum(u, 0.0)
    o_ref[...] = jnp.dot(u, w2_ref[...], preferred_element_type=f32) + b2_ref[...]


def _attn_update(h, gcol, grow, agg0, agg1, u1h, u1a, u1c, b1, w2, b2):
    return pl.pallas_call(
        _attn_update_body,
        grid=(N_PAD // _ROWS,),
        in_specs=[_const2d((N_PAD, NSD)),
                  _const2d((1, N_PAD)),
                  pl.BlockSpec((_ROWS, 1), lambda i: (i, 0)),
                  pl.BlockSpec((_ROWS, MSG), lambda i: (i, 0)),
                  pl.BlockSpec((_ROWS, MSG), lambda i: (i, 0)),
                  _const2d((NSD, 64)), _const2d((MSG, 64)),
                  _const2d((NSD, 64)), _const2d((1, 64)),
                  _const2d((64, NSD)), _const2d((1, NSD))],
        out_specs=pl.BlockSpec((_ROWS, NSD), lambda i: (i, 0)),
        out_shape=jax.ShapeDtypeStruct((N_PAD, NSD), jnp.float32),
    )(h, gcol, grow, agg0, agg1, u1h, u1a, u1c, b1, w2, b2)


def _readout_body(h_ref, grow_ref, wg_ref, bg_ref, wv_ref, bv_ref,
                  w1a_ref, w1b_ref, b1_ref, w2_ref, b2_ref, se_ref, so_ref,
                  rep_ref, out_ref):
    f32 = jnp.float32
    i = pl.program_id(0)
    hb = h_ref[...]
    gate_z = jnp.dot(hb, wg_ref[...], preferred_element_type=f32) + bg_ref[...]
    gate = 1.0 / (1.0 + jnp.exp(-gate_z))
    val = jnp.dot(hb, wv_ref[...], preferred_element_type=f32) + bv_ref[...]
    gv = gate * val
    gid = lax.broadcasted_iota(jnp.int32, (1, N_GRAPHS), 1)
    onehot = (grow_ref[...] == gid).astype(f32)
    contrib = lax.dot_general(onehot, gv, (((0,), (0,)), ((), ())),
                              preferred_element_type=f32)

    @pl.when(i == 0)
    def _():
        rep_ref[...] = contrib

    @pl.when(i > 0)
    def _():
        rep_ref[...] = rep_ref[...] + contrib

    @pl.when(i == pl.num_programs(0) - 1)
    def _():
        rep = rep_ref[...]
        t = (jnp.dot(se_ref[...],
                     jnp.dot(rep, w1a_ref[...], preferred_element_type=f32),
                     preferred_element_type=f32)
             + jnp.dot(so_ref[...],
                       jnp.dot(rep, w1b_ref[...], preferred_element_type=f32),
                       preferred_element_type=f32)
             + b1_ref[...])
        t = jnp.maximum(t, 0.0)
        out_ref[...] = jnp.dot(t, w2_ref[...], preferred_element_type=f32) + b2_ref[...]


def _readout(h, grow, wg, bg, wv, bv, w1a, w1b, b1, w2, b2, se, so):
    n_pairs = N_GRAPHS // 2
    blk = 1024
    return pl.pallas_call(
        _readout_body,
        grid=(N_PAD // blk,),
        in_specs=[pl.BlockSpec((blk, NSD), lambda i: (i, 0)),
                  pl.BlockSpec((blk, 1), lambda i: (i, 0)),
                  _const2d((NSD, GREP)), _const2d((1, GREP)),
                  _const2d((NSD, GREP)), _const2d((1, GREP)),
                  _const2d((GREP, HEAD)), _const2d((GREP, HEAD)),
                  _const2d((1, HEAD)), _const2d((HEAD, 1)), _const2d((1, 1)),
                  _const2d((n_pairs, N_GRAPHS)), _const2d((n_pairs, N_GRAPHS))],
        out_specs=[_const2d((N_GRAPHS, GREP)),
                   _const2d((n_pairs, 1))],
        out_shape=[jax.ShapeDtypeStruct((N_GRAPHS, GREP), jnp.float32),
                   jax.ShapeDtypeStruct((n_pairs, 1), jnp.float32)],
    )(h, grow, wg, bg, wv, bv, w1a, w1b, b1, w2, b2, se, so)


def kernel(node_features, edge_features, from_idx, to_idx, graph_idx,
           graph_idx_4edge, training_n_graphs_in_batch,
           W_ne, b_ne, W_ee, b_ee, W_m1, b_m1, W_m2, b_m2, W_eu, b_eu,
           W_u1, b_u1, W_u2, b_u2, W_g, b_g, W_v, b_v,
           W_fc1, b_fc1, W_fc2, b_fc2):
    f32 = jnp.float32
    ep = E_PAD - N_EDGES
    nf = jnp.pad(node_features, ((0, N_PAD - N_NODES), (0, 0)))
    ef = jnp.pad(edge_features, ((0, ep), (0, 0)))
    fi = jnp.pad(from_idx, (0, ep), constant_values=N_NODES)
    ti = jnp.pad(to_idx, (0, ep), constant_values=N_NODES)
    gi = jnp.pad(graph_idx, (0, N_PAD - N_NODES), constant_values=N_GRAPHS)
    gcol = gi.reshape(1, N_PAD)
    grow = gi.reshape(N_PAD, 1)
    ti2d = ti.reshape(E_PAD // _CHUNK, _CHUNK)
    zeros_nm = jnp.zeros((N_PAD, MSG), f32)

    a1, a2, c1 = W_m1[:NSD], W_m1[NSD:2 * NSD], W_m1[2 * NSD:]
    ae, be, ce = W_eu[:NSD], W_eu[NSD:2 * NSD], W_eu[2 * NSD:]
    u1h, u1a, u1c = W_u1[:NSD], W_u1[NSD:NSD + MSG], W_u1[NSD + MSG:]
    w1a, w1b = W_fc1[:GREP], W_fc1[GREP:]
    b_ne2 = b_ne.reshape(1, -1)
    b_ee2 = b_ee.reshape(1, -1)
    b_m12 = b_m1.reshape(1, -1)
    b_m22 = b_m2.reshape(1, -1)
    b_eu2 = b_eu.reshape(1, -1)
    b_u12 = b_u1.reshape(1, -1)
    b_u22 = b_u2.reshape(1, -1)
    b_g2 = b_g.reshape(1, -1)
    b_v2 = b_v.reshape(1, -1)
    b_fc12 = b_fc1.reshape(1, -1)
    b_fc22 = b_fc2.reshape(1, -1)
    eye16 = jnp.eye(N_GRAPHS, dtype=f32)
    se, so = eye16[0::2], eye16[1::2]
    eye_e = jnp.eye(ESD, dtype=f32)
    zero_e = jnp.zeros((1, ESD), f32)

    h = _node_encoder(nf, W_ne, b_ne2)
    e = ef
    wee, bee = W_ee, b_ee2
    for _ in range(2):
        src, dst = _gather_edges(h, fi, ti)
        msg, e = _edge_mlp(src, dst, e, wee, bee, a1, a2, c1, b_m12,
                           W_m2, b_m22, ae, be, ce, b_eu2)
        wee, bee = eye_e, zero_e
        parts = _scatter_msg(msg, ti2d, zeros_nm)
        h = _attn_update(h, gcol, grow, parts[0], parts[1],
                         u1h, u1a, u1c, b_u12, W_u2, b_u22)

    _, out = _readout(h, grow, W_g, b_g2, W_v, b_v2,
                      w1a, w1b, b_fc12, W_fc2, b_fc22, se, so)
    return out[:, 0]


# speedup vs baseline: 1.3190x; 1.1010x over previous
import functools

import jax
import jax.numpy as jnp
from jax import lax
from jax.experimental import pallas as pl
from jax.experimental.pallas import tpu as pltpu
from jax.experimental.pallas import tpu_sc as plsc

N_NODES = 10000
N_PAD = 10240
N_EDGES = 320000
NSD = 32
ESD = 16
MSG = 32
GREP = 128
HEAD = 256
N_GRAPHS = 16

_NW = 32
_CHUNK = 128
_CPW = 80
E_PAD = _NW * _CPW * _CHUNK


@functools.cache
def _build_gather():
    mesh = plsc.VectorSubcoreMesh(
        core_axis_name="c", subcore_axis_name="s", num_cores=2, num_subcores=16)

    @functools.partial(
        pl.kernel,
        out_type=(jax.ShapeDtypeStruct((E_PAD, NSD), jnp.float32),
                  jax.ShapeDtypeStruct((E_PAD, NSD), jnp.float32)),
        mesh=mesh,
        scratch_types=[
            pltpu.VMEM((_CHUNK,), jnp.int32),
            pltpu.VMEM((_CHUNK,), jnp.int32),
            pltpu.VMEM((_CHUNK, NSD), jnp.float32),
            pltpu.VMEM((_CHUNK, NSD), jnp.float32),
            pltpu.VMEM_SHARED((N_PAD, NSD), jnp.float32),
            pltpu.SemaphoreType.DMA,
            pltpu.SemaphoreType.DMA,
        ],
        compiler_params=pltpu.CompilerParams(use_tc_tiling_on_sc=False),
    )
    def gather_edges(h_hbm, fidx_hbm, tidx_hbm, src_out, dst_out,
                     fidx_v, tidx_v, srow_v, trow_v, h_sh, sem_f, sem_t):
        cid = lax.axis_index("c")
        sid = lax.axis_index("s")
        wid = sid * 2 + cid

        @pl.when(sid == 0)
        def _():
            pltpu.sync_copy(h_hbm, h_sh)

        plsc.subcore_barrier()

        def body(j, carry):
            base = (wid * _CPW + j) * _CHUNK
            pltpu.sync_copy(fidx_hbm.at[pl.ds(base, _CHUNK)], fidx_v)
            pltpu.sync_copy(tidx_hbm.at[pl.ds(base, _CHUNK)], tidx_v)
            cp_f = pltpu.async_copy(h_sh.at[fidx_v], srow_v, sem_f)
            cp_t = pltpu.async_copy(h_sh.at[tidx_v], trow_v, sem_t)
            cp_f.wait()
            cp_t.wait()
            pltpu.sync_copy(srow_v, src_out.at[pl.ds(base, _CHUNK)])
            pltpu.sync_copy(trow_v, dst_out.at[pl.ds(base, _CHUNK)])
            return carry

        lax.fori_loop(0, _CPW, body, 0)

    return gather_edges


def _gather_edges(h, fi, ti):
    return _build_gather()(h, fi, ti)


@functools.cache
def _build_scatter():
    mesh = plsc.VectorSubcoreMesh(
        core_axis_name="c", subcore_axis_name="s", num_cores=2, num_subcores=16)

    @functools.partial(
        pl.kernel,
        out_type=jax.ShapeDtypeStruct((2, N_PAD, MSG), jnp.float32),
        mesh=mesh,
        scratch_types=[
            pltpu.VMEM((_CPW, _CHUNK), jnp.int32),
            pltpu.VMEM((_CHUNK, MSG), jnp.float32),
            pltpu.VMEM_SHARED((N_PAD, MSG), jnp.float32),
        ],
        compiler_params=pltpu.CompilerParams(use_tc_tiling_on_sc=False),
    )
    def scatter_msg(msg_hbm, tidx2d_hbm, zeros_hbm, out_hbm, idx_v, msg_v, acc_sh):
        cid = lax.axis_index("c")
        sid = lax.axis_index("s")
        wid = sid * 2 + cid

        @pl.when(sid == 0)
        def _():
            pltpu.sync_copy(zeros_hbm, acc_sh)

        plsc.subcore_barrier()
        pltpu.sync_copy(tidx2d_hbm.at[pl.ds(wid * _CPW, _CPW)], idx_v)

        def body(j, carry):
            base = (wid * _CPW + j) * _CHUNK
            pltpu.sync_copy(msg_hbm.at[pl.ds(base, _CHUNK)], msg_v)
            pltpu.sync_copy(msg_v, acc_sh.at[idx_v.at[j]], add=True)
            return carry

        lax.fori_loop(0, _CPW, body, 0)
        plsc.subcore_barrier()

        @pl.when(sid == 0)
        def _():
            pltpu.sync_copy(acc_sh, out_hbm.at[cid])

    return scatter_msg


def _scatter_msg(msg, ti2d, zeros_nm):
    return _build_scatter()(msg, ti2d, zeros_nm)


def _const2d(shape):
    return pl.BlockSpec(shape, lambda i: (0, 0))


def _encoder_body(x_ref, w_ref, b_ref, o_ref):
    o_ref[...] = jnp.dot(x_ref[...], w_ref[...],
                         preferred_element_type=jnp.float32) + b_ref[...]


def _node_encoder(x, w, b):
    blk = 2048
    return pl.pallas_call(
        _encoder_body,
        grid=(N_PAD // blk,),
        in_specs=[pl.BlockSpec((blk, 128), lambda i: (i, 0)),
                  _const2d((128, NSD)), _const2d((1, NSD))],
        out_specs=pl.BlockSpec((blk, NSD), lambda i: (i, 0)),
        out_shape=jax.ShapeDtypeStruct((N_PAD, NSD), jnp.float32),
    )(x, w, b)


def _edge_mlp_body(src_ref, dst_ref, x_ref, wee_ref, bee_ref,
                   a1_ref, a2_ref, c1_ref, b1_ref, w2_ref, b2_ref,
                   ae_ref, be_ref, ce_ref, beu_ref, msg_ref, eo_ref):
    f32 = jnp.float32
    src = src_ref[...]
    dst = dst_ref[...]
    e0 = jnp.dot(x_ref[...], wee_ref[...], preferred_element_type=f32) + bee_ref[...]
    hid = (jnp.dot(src, a1_ref[...], preferred_element_type=f32)
           + jnp.dot(dst, a2_ref[...], preferred_element_type=f32)
           + jnp.dot(e0, c1_ref[...], preferred_element_type=f32)
           + b1_ref[...])
    hid = jnp.maximum(hid, 0.0)
    msg_ref[...] = jnp.dot(hid, w2_ref[...], preferred_element_type=f32) + b2_ref[...]
    eo = (jnp.dot(src, ae_ref[...], preferred_element_type=f32)
          + jnp.dot(dst, be_ref[...], preferred_element_type=f32)
          + jnp.dot(e0, ce_ref[...], preferred_element_type=f32)
          + beu_ref[...])
    eo_ref[...] = jnp.maximum(eo, 0.0)


def _edge_mlp(src, dst, x, wee, bee, a1, a2, c1, b1, w2, b2, ae, be, ce, beu):
    blk = 8192
    return pl.pallas_call(
        _edge_mlp_body,
        grid=(E_PAD // blk,),
        in_specs=[pl.BlockSpec((blk, NSD), lambda i: (i, 0)),
                  pl.BlockSpec((blk, NSD), lambda i: (i, 0)),
                  pl.BlockSpec((blk, ESD), lambda i: (i, 0)),
                  _const2d((ESD, ESD)), _const2d((1, ESD)),
                  _const2d((NSD, 64)), _const2d((NSD, 64)),
                  _const2d((ESD, 64)), _const2d((1, 64)),
                  _const2d((64, MSG)), _const2d((1, MSG)),
                  _const2d((NSD, ESD)), _const2d((NSD, ESD)),
                  _const2d((ESD, ESD)), _const2d((1, ESD))],
        out_specs=[pl.BlockSpec((blk, MSG), lambda i: (i, 0)),
                   pl.BlockSpec((blk, ESD), lambda i: (i, 0))],
        out_shape=[jax.ShapeDtypeStruct((E_PAD, MSG), jnp.float32),
                   jax.ShapeDtypeStruct((E_PAD, ESD), jnp.float32)],
    )(src, dst, x, wee, bee, a1, a2, c1, b1, w2, b2, ae, be, ce, beu)


_ROWS = 128


_CTILE = 512


def _attn_update_body(tlo_ref, thi_ref, h_ref, gcol_ref, grow_ref,
                      agg0_ref, agg1_ref,
                      u1h_ref, u1a_ref, u1c_ref, b1_ref, w2_ref, b2_ref,
                      o_ref):
    f32 = jnp.float32
    i = pl.program_id(0)
    hr = h_ref[pl.ds(i * _ROWS, _ROWS), :]
    partner = grow_ref[...] ^ 1

    def tile(t, carry):
        m, l, acc = carry
        cols = pl.ds(t * _CTILE, _CTILE)
        hc = h_ref[cols, :]
        s = lax.dot_general(hr, hc, (((1,), (1,)), ((), ())),
                            preferred_element_type=f32)
        mask = partner == gcol_ref[:, cols]
        s = jnp.where(mask, s, jnp.float32(-1e9))
        m_new = jnp.maximum(m, jnp.max(s, axis=1, keepdims=True))
        p = jnp.exp(s - m_new)
        scale = jnp.exp(m - m_new)
        l = l * scale + jnp.sum(p, axis=1, keepdims=True)
        acc = acc * scale + lax.dot_general(p, hc, (((1,), (0,)), ((), ())),
                                            preferred_element_type=f32)
        return m_new, l, acc

    m0 = jnp.full((_ROWS, 1), -1e9, f32)
    l0 = jnp.zeros((_ROWS, 1), f32)
    a0 = jnp.zeros((_ROWS, NSD), f32)
    _, l, acc = lax.fori_loop(tlo_ref[i], thi_ref[i], tile, (m0, l0, a0))
    cross = hr - acc / l
    agg = agg0_ref[...] + agg1_ref[...]
    u = (jnp.dot(hr, u1h_ref[...], preferred_element_type=f32)
         + jnp.dot(agg, u1a_ref[...], preferred_element_type=f32)
         + jnp.dot(cross, u1c_ref[...], preferred_element_type=f32)
         + b1_ref[...])
    u = jnp.maximum(u, 0.0)
    o_ref[...] = jnp.dot(u, w2_ref[...], preferred_element_type=f32) + b2_ref[...]


def _attn_update(tlo, thi, h, gcol, grow, agg0, agg1, u1h, u1a, u1c, b1, w2, b2):
    smem = pl.BlockSpec(memory_space=pltpu.SMEM)
    return pl.pallas_call(
        _attn_update_body,
        grid=(N_PAD // _ROWS,),
        in_specs=[smem, smem,
                  _const2d((N_PAD, NSD)),
                  _const2d((1, N_PAD)),
                  pl.BlockSpec((_ROWS, 1), lambda i: (i, 0)),
                  pl.BlockSpec((_ROWS, MSG), lambda i: (i, 0)),
                  pl.BlockSpec((_ROWS, MSG), lambda i: (i, 0)),
                  _const2d((NSD, 64)), _const2d((MSG, 64)),
                  _const2d((NSD, 64)), _const2d((1, 64)),
                  _const2d((64, NSD)), _const2d((1, NSD))],
        out_specs=pl.BlockSpec((_ROWS, NSD), lambda i: (i, 0)),
        out_shape=jax.ShapeDtypeStruct((N_PAD, NSD), jnp.float32),
    )(tlo, thi, h, gcol, grow, agg0, agg1, u1h, u1a, u1c, b1, w2, b2)


def _readout_body(h_ref, grow_ref, wg_ref, bg_ref, wv_ref, bv_ref,
                  w1a_ref, w1b_ref, b1_ref, w2_ref, b2_ref, se_ref, so_ref,
                  rep_ref, out_ref):
    f32 = jnp.float32
    i = pl.program_id(0)
    hb = h_ref[...]
    gate_z = jnp.dot(hb, wg_ref[...], preferred_element_type=f32) + bg_ref[...]
    gate = 1.0 / (1.0 + jnp.exp(-gate_z))
    val = jnp.dot(hb, wv_ref[...], preferred_element_type=f32) + bv_ref[...]
    gv = gate * val
    gid = lax.broadcasted_iota(jnp.int32, (1, N_GRAPHS), 1)
    onehot = (grow_ref[...] == gid).astype(f32)
    contrib = lax.dot_general(onehot, gv, (((0,), (0,)), ((), ())),
                              preferred_element_type=f32)

    @pl.when(i == 0)
    def _():
        rep_ref[...] = contrib

    @pl.when(i > 0)
    def _():
        rep_ref[...] = rep_ref[...] + contrib

    @pl.when(i == pl.num_programs(0) - 1)
    def _():
        rep = rep_ref[...]
        t = (jnp.dot(se_ref[...],
                     jnp.dot(rep, w1a_ref[...], preferred_element_type=f32),
                     preferred_element_type=f32)
             + jnp.dot(so_ref[...],
                       jnp.dot(rep, w1b_ref[...], preferred_element_type=f32),
                       preferred_element_type=f32)
             + b1_ref[...])
        t = jnp.maximum(t, 0.0)
        out_ref[...] = jnp.dot(t, w2_ref[...], preferred_element_type=f32) + b2_ref[...]


def _readout(h, grow, wg, bg, wv, bv, w1a, w1b, b1, w2, b2, se, so):
    n_pairs = N_GRAPHS // 2
    blk = 1024
    return pl.pallas_call(
        _readout_body,
        grid=(N_PAD // blk,),
        in_specs=[pl.BlockSpec((blk, NSD), lambda i: (i, 0)),
                  pl.BlockSpec((blk, 1), lambda i: (i, 0)),
                  _const2d((NSD, GREP)), _const2d((1, GREP)),
                  _const2d((NSD, GREP)), _const2d((1, GREP)),
                  _const2d((GREP, HEAD)), _const2d((GREP, HEAD)),
                  _const2d((1, HEAD)), _const2d((HEAD, 1)), _const2d((1, 1)),
                  _const2d((n_pairs, N_GRAPHS)), _const2d((n_pairs, N_GRAPHS))],
        out_specs=[_const2d((N_GRAPHS, GREP)),
                   _const2d((n_pairs, 1))],
        out_shape=[jax.ShapeDtypeStruct((N_GRAPHS, GREP), jnp.float32),
                   jax.ShapeDtypeStruct((n_pairs, 1), jnp.float32)],
    )(h, grow, wg, bg, wv, bv, w1a, w1b, b1, w2, b2, se, so)


def kernel(node_features, edge_features, from_idx, to_idx, graph_idx,
           graph_idx_4edge, training_n_graphs_in_batch,
           W_ne, b_ne, W_ee, b_ee, W_m1, b_m1, W_m2, b_m2, W_eu, b_eu,
           W_u1, b_u1, W_u2, b_u2, W_g, b_g, W_v, b_v,
           W_fc1, b_fc1, W_fc2, b_fc2):
    f32 = jnp.float32
    ep = E_PAD - N_EDGES
    nf = jnp.pad(node_features, ((0, N_PAD - N_NODES), (0, 0)))
    ef = jnp.pad(edge_features, ((0, ep), (0, 0)))
    fi = jnp.pad(from_idx, (0, ep), constant_values=N_NODES)
    ti = jnp.pad(to_idx, (0, ep), constant_values=N_NODES)
    gi = jnp.pad(graph_idx, (0, N_PAD - N_NODES), constant_values=N_GRAPHS)
    gcol = gi.reshape(1, N_PAD)
    grow = gi.reshape(N_PAD, 1)
    pair = gi // 2
    lo = jnp.searchsorted(pair, pair[0::_ROWS], side='left').astype(jnp.int32)
    hi = jnp.searchsorted(pair, pair[_ROWS - 1::_ROWS], side='right').astype(jnp.int32)
    tlo = lo // _CTILE
    thi = (hi + _CTILE - 1) // _CTILE
    ti2d = ti.reshape(E_PAD // _CHUNK, _CHUNK)
    zeros_nm = jnp.zeros((N_PAD, MSG), f32)

    a1, a2, c1 = W_m1[:NSD], W_m1[NSD:2 * NSD], W_m1[2 * NSD:]
    ae, be, ce = W_eu[:NSD], W_eu[NSD:2 * NSD], W_eu[2 * NSD:]
    u1h, u1a, u1c = W_u1[:NSD], W_u1[NSD:NSD + MSG], W_u1[NSD + MSG:]
    w1a, w1b = W_fc1[:GREP], W_fc1[GREP:]
    b_ne2 = b_ne.reshape(1, -1)
    b_ee2 = b_ee.reshape(1, -1)
    b_m12 = b_m1.reshape(1, -1)
    b_m22 = b_m2.reshape(1, -1)
    b_eu2 = b_eu.reshape(1, -1)
    b_u12 = b_u1.reshape(1, -1)
    b_u22 = b_u2.reshape(1, -1)
    b_g2 = b_g.reshape(1, -1)
    b_v2 = b_v.reshape(1, -1)
    b_fc12 = b_fc1.reshape(1, -1)
    b_fc22 = b_fc2.reshape(1, -1)
    eye16 = jnp.eye(N_GRAPHS, dtype=f32)
    se, so = eye16[0::2], eye16[1::2]
    eye_e = jnp.eye(ESD, dtype=f32)
    zero_e = jnp.zeros((1, ESD), f32)

    h = _node_encoder(nf, W_ne, b_ne2)
    e = ef
    wee, bee = W_ee, b_ee2
    for _ in range(2):
        src, dst = _gather_edges(h, fi, ti)
        msg, e = _edge_mlp(src, dst, e, wee, bee, a1, a2, c1, b_m12,
                           W_m2, b_m22, ae, be, ce, b_eu2)
        wee, bee = eye_e, zero_e
        parts = _scatter_msg(msg, ti2d, zeros_nm)
        h = _attn_update(tlo, thi, h, gcol, grow, parts[0], parts[1],
                         u1h, u1a, u1c, b_u12, W_u2, b_u22)

    _, out = _readout(h, grow, W_g, b_g2, W_v, b_v2,
                      w1a, w1b, b_fc12, W_fc2, b_fc22, se, so)
    return out[:, 0]


# speedup vs baseline: 1.4153x; 1.0730x over previous
import functools

import jax
import jax.numpy as jnp
from jax import lax
from jax.experimental import pallas as pl
from jax.experimental.pallas import tpu as pltpu
from jax.experimental.pallas import tpu_sc as plsc

N_NODES = 10000
N_PAD = 10240
N_EDGES = 320000
NSD = 32
ESD = 16
MSG = 32
GREP = 128
HEAD = 256
N_GRAPHS = 16

_NW = 32
_CHUNK = 128
_CPW = 80
_GB = 8
E_PAD = _NW * _CPW * _CHUNK


@functools.cache
def _build_gather():
    mesh = plsc.VectorSubcoreMesh(
        core_axis_name="c", subcore_axis_name="s", num_cores=2, num_subcores=16)

    @functools.partial(
        pl.kernel,
        out_type=(jax.ShapeDtypeStruct((E_PAD, NSD), jnp.float32),
                  jax.ShapeDtypeStruct((E_PAD, NSD), jnp.float32)),
        mesh=mesh,
        scratch_types=[
            pltpu.VMEM((_GB, _CHUNK), jnp.int32),
            pltpu.VMEM((_GB, _CHUNK), jnp.int32),
            pltpu.VMEM((_GB * _CHUNK, NSD), jnp.float32),
            pltpu.VMEM((_GB * _CHUNK, NSD), jnp.float32),
            pltpu.VMEM_SHARED((N_PAD, NSD), jnp.float32),
            pltpu.SemaphoreType.DMA,
            pltpu.SemaphoreType.DMA,
        ],
        compiler_params=pltpu.CompilerParams(use_tc_tiling_on_sc=False),
    )
    def gather_edges(h_hbm, fidx2d_hbm, tidx2d_hbm, src_out, dst_out,
                     fidx_v, tidx_v, srow_v, trow_v, h_sh, sem_f, sem_t):
        cid = lax.axis_index("c")
        sid = lax.axis_index("s")
        wid = sid * 2 + cid

        @pl.when(sid == 0)
        def _():
            pltpu.sync_copy(h_hbm, h_sh)

        plsc.subcore_barrier()

        def body(j, carry):
            row0 = wid * _CPW + j * _GB
            base = row0 * _CHUNK
            pltpu.sync_copy(fidx2d_hbm.at[pl.ds(row0, _GB)], fidx_v)
            pltpu.sync_copy(tidx2d_hbm.at[pl.ds(row0, _GB)], tidx_v)
            cps = []
            for k in range(_GB):
                cps.append(pltpu.async_copy(
                    h_sh.at[fidx_v.at[k]],
                    srow_v.at[pl.ds(k * _CHUNK, _CHUNK)], sem_f))
                cps.append(pltpu.async_copy(
                    h_sh.at[tidx_v.at[k]],
                    trow_v.at[pl.ds(k * _CHUNK, _CHUNK)], sem_t))
            for cp in cps:
                cp.wait()
            pltpu.sync_copy(srow_v, src_out.at[pl.ds(base, _GB * _CHUNK)])
            pltpu.sync_copy(trow_v, dst_out.at[pl.ds(base, _GB * _CHUNK)])
            return carry

        lax.fori_loop(0, _CPW // _GB, body, 0)

    return gather_edges


def _gather_edges(h, fi2d, ti2d):
    return _build_gather()(h, fi2d, ti2d)


@functools.cache
def _build_scatter():
    mesh = plsc.VectorSubcoreMesh(
        core_axis_name="c", subcore_axis_name="s", num_cores=2, num_subcores=16)

    @functools.partial(
        pl.kernel,
        out_type=jax.ShapeDtypeStruct((2, N_PAD, MSG), jnp.float32),
        mesh=mesh,
        scratch_types=[
            pltpu.VMEM((_CPW, _CHUNK), jnp.int32),
            pltpu.VMEM((_CHUNK, MSG), jnp.float32),
            pltpu.VMEM_SHARED((N_PAD, MSG), jnp.float32),
        ],
        compiler_params=pltpu.CompilerParams(use_tc_tiling_on_sc=False),
    )
    def scatter_msg(msg_hbm, tidx2d_hbm, zeros_hbm, out_hbm, idx_v, msg_v, acc_sh):
        cid = lax.axis_index("c")
        sid = lax.axis_index("s")
        wid = sid * 2 + cid

        @pl.when(sid == 0)
        def _():
            pltpu.sync_copy(zeros_hbm, acc_sh)

        plsc.subcore_barrier()
        pltpu.sync_copy(tidx2d_hbm.at[pl.ds(wid * _CPW, _CPW)], idx_v)

        def body(j, carry):
            base = (wid * _CPW + j) * _CHUNK
            pltpu.sync_copy(msg_hbm.at[pl.ds(base, _CHUNK)], msg_v)
            pltpu.sync_copy(msg_v, acc_sh.at[idx_v.at[j]], add=True)
            return carry

        lax.fori_loop(0, _CPW, body, 0)
        plsc.subcore_barrier()

        @pl.when(sid == 0)
        def _():
            pltpu.sync_copy(acc_sh, out_hbm.at[cid])

    return scatter_msg


def _scatter_msg(msg, ti2d, zeros_nm):
    return _build_scatter()(msg, ti2d, zeros_nm)


def _const2d(shape):
    return pl.BlockSpec(shape, lambda i: (0, 0))


def _encoder_body(x_ref, w_ref, b_ref, o_ref):
    o_ref[...] = jnp.dot(x_ref[...], w_ref[...],
                         preferred_element_type=jnp.float32) + b_ref[...]


def _node_encoder(x, w, b):
    blk = 2048
    return pl.pallas_call(
        _encoder_body,
        grid=(N_PAD // blk,),
        in_specs=[pl.BlockSpec((blk, 128), lambda i: (i, 0)),
                  _const2d((128, NSD)), _const2d((1, NSD))],
        out_specs=pl.BlockSpec((blk, NSD), lambda i: (i, 0)),
        out_shape=jax.ShapeDtypeStruct((N_PAD, NSD), jnp.float32),
    )(x, w, b)


def _edge_mlp_body(src_ref, dst_ref, x_ref, wee_ref, bee_ref,
                   a1_ref, a2_ref, c1_ref, b1_ref, w2_ref, b2_ref,
                   ae_ref, be_ref, ce_ref, beu_ref, msg_ref, eo_ref):
    f32 = jnp.float32
    src = src_ref[...]
    dst = dst_ref[...]
    e0 = jnp.dot(x_ref[...], wee_ref[...], preferred_element_type=f32) + bee_ref[...]
    hid = (jnp.dot(src, a1_ref[...], preferred_element_type=f32)
           + jnp.dot(dst, a2_ref[...], preferred_element_type=f32)
           + jnp.dot(e0, c1_ref[...], preferred_element_type=f32)
           + b1_ref[...])
    hid = jnp.maximum(hid, 0.0)
    msg_ref[...] = jnp.dot(hid, w2_ref[...], preferred_element_type=f32) + b2_ref[...]
    eo = (jnp.dot(src, ae_ref[...], preferred_element_type=f32)
          + jnp.dot(dst, be_ref[...], preferred_element_type=f32)
          + jnp.dot(e0, ce_ref[...], preferred_element_type=f32)
          + beu_ref[...])
    eo_ref[...] = jnp.maximum(eo, 0.0)


def _edge_mlp(src, dst, x, wee, bee, a1, a2, c1, b1, w2, b2, ae, be, ce, beu):
    blk = 8192
    return pl.pallas_call(
        _edge_mlp_body,
        grid=(E_PAD // blk,),
        in_specs=[pl.BlockSpec((blk, NSD), lambda i: (i, 0)),
                  pl.BlockSpec((blk, NSD), lambda i: (i, 0)),
                  pl.BlockSpec((blk, ESD), lambda i: (i, 0)),
                  _const2d((ESD, ESD)), _const2d((1, ESD)),
                  _const2d((NSD, 64)), _const2d((NSD, 64)),
                  _const2d((ESD, 64)), _const2d((1, 64)),
                  _const2d((64, MSG)), _const2d((1, MSG)),
                  _const2d((NSD, ESD)), _const2d((NSD, ESD)),
                  _const2d((ESD, ESD)), _const2d((1, ESD))],
        out_specs=[pl.BlockSpec((blk, MSG), lambda i: (i, 0)),
                   pl.BlockSpec((blk, ESD), lambda i: (i, 0))],
        out_shape=[jax.ShapeDtypeStruct((E_PAD, MSG), jnp.float32),
                   jax.ShapeDtypeStruct((E_PAD, ESD), jnp.float32)],
    )(src, dst, x, wee, bee, a1, a2, c1, b1, w2, b2, ae, be, ce, beu)


_ROWS = 128


_CTILE = 512


def _attn_update_body(tlo_ref, thi_ref, h_ref, gcol_ref, grow_ref,
                      agg0_ref, agg1_ref,
                      u1h_ref, u1a_ref, u1c_ref, b1_ref, w2_ref, b2_ref,
                      o_ref):
    f32 = jnp.float32
    i = pl.program_id(0)
    hr = h_ref[pl.ds(i * _ROWS, _ROWS), :]
    partner = grow_ref[...] ^ 1

    def tile(t, carry):
        m, l, acc = carry
        cols = pl.ds(t * _CTILE, _CTILE)
        hc = h_ref[cols, :]
        s = lax.dot_general(hr, hc, (((1,), (1,)), ((), ())),
                            preferred_element_type=f32)
        mask = partner == gcol_ref[:, cols]
        s = jnp.where(mask, s, jnp.float32(-1e9))
        m_new = jnp.maximum(m, jnp.max(s, axis=1, keepdims=True))
        p = jnp.exp(s - m_new)
        scale = jnp.exp(m - m_new)
        l = l * scale + jnp.sum(p, axis=1, keepdims=True)
        acc = acc * scale + lax.dot_general(p, hc, (((1,), (0,)), ((), ())),
                                            preferred_element_type=f32)
        return m_new, l, acc

    m0 = jnp.full((_ROWS, 1), -1e9, f32)
    l0 = jnp.zeros((_ROWS, 1), f32)
    a0 = jnp.zeros((_ROWS, NSD), f32)
    _, l, acc = lax.fori_loop(tlo_ref[i], thi_ref[i], tile, (m0, l0, a0))
    cross = hr - acc / l
    agg = agg0_ref[...] + agg1_ref[...]
    u = (jnp.dot(hr, u1h_ref[...], preferred_element_type=f32)
         + jnp.dot(agg, u1a_ref[...], preferred_element_type=f32)
         + jnp.dot(cross, u1c_ref[...], preferred_element_type=f32)
         + b1_ref[...])
    u = jnp.maximum(u, 0.0)
    o_ref[...] = jnp.dot(u, w2_ref[...], preferred_element_type=f32) + b2_ref[...]


def _attn_update(tlo, thi, h, gcol, grow, agg0, agg1, u1h, u1a, u1c, b1, w2, b2):
    smem = pl.BlockSpec(memory_space=pltpu.SMEM)
    return pl.pallas_call(
        _attn_update_body,
        grid=(N_PAD // _ROWS,),
        in_specs=[smem, smem,
                  _const2d((N_PAD, NSD)),
                  _const2d((1, N_PAD)),
                  pl.BlockSpec((_ROWS, 1), lambda i: (i, 0)),
                  pl.BlockSpec((_ROWS, MSG), lambda i: (i, 0)),
                  pl.BlockSpec((_ROWS, MSG), lambda i: (i, 0)),
                  _const2d((NSD, 64)), _const2d((MSG, 64)),
                  _const2d((NSD, 64)), _const2d((1, 64)),
                  _const2d((64, NSD)), _const2d((1, NSD))],
        out_specs=pl.BlockSpec((_ROWS, NSD), lambda i: (i, 0)),
        out_shape=jax.ShapeDtypeStruct((N_PAD, NSD), jnp.float32),
    )(tlo, thi, h, gcol, grow, agg0, agg1, u1h, u1a, u1c, b1, w2, b2)


def _readout_body(h_ref, grow_ref, wg_ref, bg_ref, wv_ref, bv_ref,
                  w1a_ref, w1b_ref, b1_ref, w2_ref, b2_ref, se_ref, so_ref,
                  rep_ref, out_ref):
    f32 = jnp.float32
    i = pl.program_id(0)
    hb = h_ref[...]
    gate_z = jnp.dot(hb, wg_ref[...], preferred_element_type=f32) + bg_ref[...]
    gate = 1.0 / (1.0 + jnp.exp(-gate_z))
    val = jnp.dot(hb, wv_ref[...], preferred_element_type=f32) + bv_ref[...]
    gv = gate * val
    gid = lax.broadcasted_iota(jnp.int32, (1, N_GRAPHS), 1)
    onehot = (grow_ref[...] == gid).astype(f32)
    contrib = lax.dot_general(onehot, gv, (((0,), (0,)), ((), ())),
                              preferred_element_type=f32)

    @pl.when(i == 0)
    def _():
        rep_ref[...] = contrib

    @pl.when(i > 0)
    def _():
        rep_ref[...] = rep_ref[...] + contrib

    @pl.when(i == pl.num_programs(0) - 1)
    def _():
        rep = rep_ref[...]
        t = (jnp.dot(se_ref[...],
                     jnp.dot(rep, w1a_ref[...], preferred_element_type=f32),
                     preferred_element_type=f32)
             + jnp.dot(so_ref[...],
                       jnp.dot(rep, w1b_ref[...], preferred_element_type=f32),
                       preferred_element_type=f32)
             + b1_ref[...])
        t = jnp.maximum(t, 0.0)
        out_ref[...] = jnp.dot(t, w2_ref[...], preferred_element_type=f32) + b2_ref[...]


def _readout(h, grow, wg, bg, wv, bv, w1a, w1b, b1, w2, b2, se, so):
    n_pairs = N_GRAPHS // 2
    blk = 1024
    return pl.pallas_call(
        _readout_body,
        grid=(N_PAD // blk,),
        in_specs=[pl.BlockSpec((blk, NSD), lambda i: (i, 0)),
                  pl.BlockSpec((blk, 1), lambda i: (i, 0)),
                  _const2d((NSD, GREP)), _const2d((1, GREP)),
                  _const2d((NSD, GREP)), _const2d((1, GREP)),
                  _const2d((GREP, HEAD)), _const2d((GREP, HEAD)),
                  _const2d((1, HEAD)), _const2d((HEAD, 1)), _const2d((1, 1)),
                  _const2d((n_pairs, N_GRAPHS)), _const2d((n_pairs, N_GRAPHS))],
        out_specs=[_const2d((N_GRAPHS, GREP)),
                   _const2d((n_pairs, 1))],
        out_shape=[jax.ShapeDtypeStruct((N_GRAPHS, GREP), jnp.float32),
                   jax.ShapeDtypeStruct((n_pairs, 1), jnp.float32)],
    )(h, grow, wg, bg, wv, bv, w1a, w1b, b1, w2, b2, se, so)


def kernel(node_features, edge_features, from_idx, to_idx, graph_idx,
           graph_idx_4edge, training_n_graphs_in_batch,
           W_ne, b_ne, W_ee, b_ee, W_m1, b_m1, W_m2, b_m2, W_eu, b_eu,
           W_u1, b_u1, W_u2, b_u2, W_g, b_g, W_v, b_v,
           W_fc1, b_fc1, W_fc2, b_fc2):
    f32 = jnp.float32
    ep = E_PAD - N_EDGES
    nf = jnp.pad(node_features, ((0, N_PAD - N_NODES), (0, 0)))
    ef = jnp.pad(edge_features, ((0, ep), (0, 0)))
    fi = jnp.pad(from_idx, (0, ep), constant_values=N_NODES)
    ti = jnp.pad(to_idx, (0, ep), constant_values=N_NODES)
    gi = jnp.pad(graph_idx, (0, N_PAD - N_NODES), constant_values=N_GRAPHS)
    gcol = gi.reshape(1, N_PAD)
    grow = gi.reshape(N_PAD, 1)
    pair = gi // 2
    lo = jnp.searchsorted(pair, pair[0::_ROWS], side='left').astype(jnp.int32)
    hi = jnp.searchsorted(pair, pair[_ROWS - 1::_ROWS], side='right').astype(jnp.int32)
    tlo = lo // _CTILE
    thi = (hi + _CTILE - 1) // _CTILE
    fi2d = fi.reshape(E_PAD // _CHUNK, _CHUNK)
    ti2d = ti.reshape(E_PAD // _CHUNK, _CHUNK)
    zeros_nm = jnp.zeros((N_PAD, MSG), f32)

    a1, a2, c1 = W_m1[:NSD], W_m1[NSD:2 * NSD], W_m1[2 * NSD:]
    ae, be, ce = W_eu[:NSD], W_eu[NSD:2 * NSD], W_eu[2 * NSD:]
    u1h, u1a, u1c = W_u1[:NSD], W_u1[NSD:NSD + MSG], W_u1[NSD + MSG:]
    w1a, w1b = W_fc1[:GREP], W_fc1[GREP:]
    b_ne2 = b_ne.reshape(1, -1)
    b_ee2 = b_ee.reshape(1, -1)
    b_m12 = b_m1.reshape(1, -1)
    b_m22 = b_m2.reshape(1, -1)
    b_eu2 = b_eu.reshape(1, -1)
    b_u12 = b_u1.reshape(1, -1)
    b_u22 = b_u2.reshape(1, -1)
    b_g2 = b_g.reshape(1, -1)
    b_v2 = b_v.reshape(1, -1)
    b_fc12 = b_fc1.reshape(1, -1)
    b_fc22 = b_fc2.reshape(1, -1)
    eye16 = jnp.eye(N_GRAPHS, dtype=f32)
    se, so = eye16[0::2], eye16[1::2]
    eye_e = jnp.eye(ESD, dtype=f32)
    zero_e = jnp.zeros((1, ESD), f32)

    h = _node_encoder(nf, W_ne, b_ne2)
    e = ef
    wee, bee = W_ee, b_ee2
    for _ in range(2):
        src, dst = _gather_edges(h, fi2d, ti2d)
        msg, e = _edge_mlp(src, dst, e, wee, bee, a1, a2, c1, b_m12,
                           W_m2, b_m22, ae, be, ce, b_eu2)
        wee, bee = eye_e, zero_e
        parts = _scatter_msg(msg, ti2d, zeros_nm)
        h = _attn_update(tlo, thi, h, gcol, grow, parts[0], parts[1],
                         u1h, u1a, u1c, b_u12, W_u2, b_u22)

    _, out = _readout(h, grow, W_g, b_g2, W_v, b_v2,
                      w1a, w1b, b_fc12, W_fc2, b_fc22, se, so)
    return out[:, 0]
